# baseline (speedup 1.0000x reference)
import functools

import jax
import jax.numpy as jnp
from jax import lax
from jax.experimental import pallas as pl
from jax.experimental.pallas import tpu as pltpu
from jax.experimental.pallas import tpu_sc as plsc

NC = 2
NS = 16
NW = NC * NS

EMBED = 64
SEQ = 50
BATCH = 1024
SAMPLE = 20
NTOK = SEQ * BATCH


def _sc_gather(table, idx, W, K):
    N = idx.shape[0]
    D = table.shape[1]
    n_per_w = N // NW
    R = n_per_w // W
    n_chunks = R // K
    assert N == NW * R * W and R == n_chunks * K

    mesh = plsc.VectorSubcoreMesh(core_axis_name="c", subcore_axis_name="s")

    @functools.partial(
        pl.kernel,
        mesh=mesh,
        out_type=jax.ShapeDtypeStruct((N, D), jnp.float32),
        compiler_params=pltpu.CompilerParams(use_tc_tiling_on_sc=False),
        scratch_types=[
            pltpu.VMEM((R, W), jnp.int32),
            pltpu.VMEM((K * W, D), jnp.float32),
            pltpu.SemaphoreType.DMA,
        ],
    )
    def gather_kernel(table_hbm, idx_hbm, out_hbm, idx_v, rows_v, sem):
        wid = lax.axis_index("s") * NC + lax.axis_index("c")
        base = wid * n_per_w
        pltpu.sync_copy(idx_hbm.at[wid], idx_v)

        def chunk_body(i, carry):
            cps = []
            for j in range(K):
                cps.append(
                    pltpu.async_copy(
                        table_hbm.at[idx_v.at[i * K + j]],
                        rows_v.at[pl.ds(j * W, W)],
                        sem,
                    )
                )
            for cp in cps:
                cp.wait()
            pltpu.sync_copy(rows_v, out_hbm.at[pl.ds(base + i * (K * W), K * W)])
            return carry

        lax.fori_loop(0, n_chunks, chunk_body, 0)

    return gather_kernel(table, idx.reshape(NW, R, W))


def _interleave_pairs(idx2d, half):
    rows = idx2d.shape[0]
    return jnp.transpose(
        idx2d.reshape(rows, -1, 2, half), (0, 1, 3, 2)
    ).reshape(rows * idx2d.shape[1])


_TBR = 512


def _transpose_body(x_ref, o_ref):
    x = x_ref[0]
    o_ref[0, :, 0:_TBR] = jnp.transpose(x[:, 0:EMBED], (1, 0))
    o_ref[0, :, _TBR : 2 * _TBR] = jnp.transpose(x[:, EMBED:128], (1, 0))


def _tc_transpose(x_lin, nj, T):
    x3 = x_lin.reshape(nj, T // 2, 128)
    n_c = (T // 2) // _TBR
    return pl.pallas_call(
        _transpose_body,
        grid=(nj, n_c),
        in_specs=[pl.BlockSpec((1, _TBR, 128), lambda j, c: (j, c, 0))],
        out_specs=pl.BlockSpec((1, EMBED, 2 * _TBR), lambda j, c: (j, 0, c)),
        out_shape=jax.ShapeDtypeStruct((nj, EMBED, T), jnp.float32),
    )(x3)


def _lstm_body(x_ref, wih_ref, whh_ref, b_ref, out_ref, h_scr, c_scr):
    t = pl.program_id(0)

    @pl.when(t == 0)
    def _init():
        h_scr[...] = jnp.zeros_like(h_scr)
        c_scr[...] = jnp.zeros_like(c_scr)

    x2 = x_ref[...]
    xt = jnp.concatenate([x2[:, 0:EMBED], x2[:, EMBED:128]], axis=0)
    gates = (
        jnp.dot(xt, wih_ref[...], preferred_element_type=jnp.float32)
        + jnp.dot(h_scr[...], whh_ref[...], preferred_element_type=jnp.float32)
        + b_ref[...]
    )
    i = jax.nn.sigmoid(gates[:, 0 * EMBED : 1 * EMBED])
    f = jax.nn.sigmoid(gates[:, 1 * EMBED : 2 * EMBED])
    g = jnp.tanh(gates[:, 2 * EMBED : 3 * EMBED])
    o = jax.nn.sigmoid(gates[:, 3 * EMBED : 4 * EMBED])
    c = f * c_scr[...] + i * g
    h = o * jnp.tanh(c)
    c_scr[...] = c
    h_scr[...] = h
    out_ref[...] = jnp.transpose(h, (1, 0))


def _lstm(x2d, wih_t, whh_t, b):
    G = 4 * EMBED
    return pl.pallas_call(
        _lstm_body,
        grid=(SEQ,),
        in_specs=[
            pl.BlockSpec((BATCH // 2, 128), lambda t: (t, 0)),
            pl.BlockSpec((EMBED, G), lambda t: (0, 0)),
            pl.BlockSpec((EMBED, G), lambda t: (0, 0)),
            pl.BlockSpec((1, G), lambda t: (0, 0)),
        ],
        out_specs=pl.BlockSpec((EMBED, BATCH), lambda t: (0, t)),
        out_shape=jax.ShapeDtypeStruct((EMBED, NTOK), jnp.float32),
        scratch_shapes=[
            pltpu.VMEM((BATCH, EMBED), jnp.float32),
            pltpu.VMEM((BATCH, EMBED), jnp.float32),
        ],
    )(x2d, wih_t, whh_t, b)


def kernel(samples, text, targets, in_embed, out_embed, W_ih, W_hh, b_ih, b_hh):
    E = in_embed.shape[1]
    sample_size = samples.shape[-1]

    txt_idx = _interleave_pairs(
        text.astype(jnp.int32), BATCH // 2
    )
    tgt_idx = _interleave_pairs(
        targets.reshape(-1, 2 * _TBR).astype(jnp.int32), _TBR
    )
    samp_idx = _interleave_pairs(
        jnp.transpose(samples, (2, 0, 1)).reshape(-1, 2 * _TBR).astype(jnp.int32),
        _TBR,
    )

    txt_emb = _sc_gather(in_embed, txt_idx, W=64, K=5)
    rnn_t = _lstm(
        txt_emb.reshape(NTOK // 2, 128),
        W_ih.T,
        W_hh.T,
        (b_ih + b_hh).reshape(1, -1),
    )
    tgt_emb = _sc_gather(out_embed, tgt_idx, W=64, K=5)
    tgt_t = _tc_transpose(tgt_emb, 1, NTOK)
    samp_emb = _sc_gather(out_embed, samp_idx, W=128, K=10)
    samp_t = _tc_transpose(samp_emb, sample_size, NTOK)

    return (
        jnp.transpose(samp_t, (2, 0, 1)),
        jnp.transpose(rnn_t, (1, 0))[:, :, None],
        jnp.transpose(tgt_t, (2, 0, 1)),
    )


# speedup vs baseline: 1.0292x; 1.0292x over previous
import functools

import jax
import jax.numpy as jnp
from jax import lax
from jax.experimental import pallas as pl
from jax.experimental.pallas import tpu as pltpu
from jax.experimental.pallas import tpu_sc as plsc

NC = 2
NS = 16
NW = NC * NS

EMBED = 64
SEQ = 50
BATCH = 1024
SAMPLE = 20
NTOK = SEQ * BATCH
HALF = 512


def _sc_gather_paired(table, idx, W, K):
    N = idx.shape[0]
    D = table.shape[1]
    n_per_w = N // NW
    R = n_per_w // W
    n_chunks = R // K
    F = K * W
    assert N == NW * R * W and R == n_chunks * K
    assert HALF % F == 0 and N % 1024 == 0

    mesh = plsc.VectorSubcoreMesh(core_axis_name="c", subcore_axis_name="s")

    @functools.partial(
        pl.kernel,
        mesh=mesh,
        out_type=jax.ShapeDtypeStruct((N // 1024, HALF, 2, D), jnp.float32),
        compiler_params=pltpu.CompilerParams(use_tc_tiling_on_sc=False),
        scratch_types=[
            pltpu.VMEM((R, W), jnp.int32),
            pltpu.VMEM((F, D), jnp.float32),
            pltpu.SemaphoreType.DMA,
        ],
    )
    def gather_kernel(table_hbm, idx_hbm, out_hbm, idx_v, rows_v, sem):
        wid = lax.axis_index("s") * NC + lax.axis_index("c")
        base = wid * n_per_w
        pltpu.sync_copy(idx_hbm.at[wid], idx_v)

        def chunk_body(i, carry):
            cps = []
            for j in range(K):
                cps.append(
                    pltpu.async_copy(
                        table_hbm.at[idx_v.at[i * K + j]],
                        rows_v.at[pl.ds(j * W, W)],
                        sem,
                    )
                )
            for cp in cps:
                cp.wait()
            a = base + i * F
            c = a // 1024
            m = a % 1024
            h = m // HALF
            rr = m % HALF
            pltpu.sync_copy(rows_v, out_hbm.at[c, pl.ds(rr, F), h])
            return carry

        lax.fori_loop(0, n_chunks, chunk_body, 0)

    return gather_kernel(table, idx.reshape(NW, R, W))


_TBR = 512


def _transpose_body(x_ref, o_ref):
    x = x_ref[0]
    eye = jnp.eye(EMBED, dtype=jnp.float32)
    dn = (((1,), (1,)), ((), ()))
    ya = lax.dot_general(eye, x[:, 0:EMBED], dn,
                         precision=lax.Precision.HIGHEST,
                         preferred_element_type=jnp.float32)
    yb = lax.dot_general(eye, x[:, EMBED:128], dn,
                         precision=lax.Precision.HIGHEST,
                         preferred_element_type=jnp.float32)
    o_ref[0, :, 0:_TBR] = ya
    o_ref[0, :, _TBR : 2 * _TBR] = yb


def _tc_transpose(x_paired, nj, T):
    x3 = x_paired.reshape(nj, T // 2, 128)
    n_c = (T // 2) // _TBR
    return pl.pallas_call(
        _transpose_body,
        grid=(nj, n_c),
        in_specs=[pl.BlockSpec((1, _TBR, 128), lambda j, c: (j, c, 0))],
        out_specs=pl.BlockSpec((1, EMBED, 2 * _TBR), lambda j, c: (j, 0, c)),
        out_shape=jax.ShapeDtypeStruct((nj, EMBED, T), jnp.float32),
    )(x3)


def _lstm_body(x_ref, wih_ref, whh_ref, b_ref, out_ref, h_scr, c_scr):
    t = pl.program_id(0)

    @pl.when(t == 0)
    def _init():
        h_scr[...] = jnp.zeros_like(h_scr)
        c_scr[...] = jnp.zeros_like(c_scr)

    x2 = x_ref[...]
    xt = jnp.concatenate([x2[:, 0:EMBED], x2[:, EMBED:128]], axis=0)
    gates = (
        jnp.dot(xt, wih_ref[...], preferred_element_type=jnp.float32)
        + jnp.dot(h_scr[...], whh_ref[...], preferred_element_type=jnp.float32)
        + b_ref[...]
    )
    i = jax.nn.sigmoid(gates[:, 0 * EMBED : 1 * EMBED])
    f = jax.nn.sigmoid(gates[:, 1 * EMBED : 2 * EMBED])
    g = jnp.tanh(gates[:, 2 * EMBED : 3 * EMBED])
    o = jax.nn.sigmoid(gates[:, 3 * EMBED : 4 * EMBED])
    c = f * c_scr[...] + i * g
    h = o * jnp.tanh(c)
    c_scr[...] = c
    h_scr[...] = h
    eye = jnp.eye(EMBED, dtype=jnp.float32)
    out_ref[...] = lax.dot_general(
        eye, h, (((1,), (1,)), ((), ())),
        precision=lax.Precision.HIGHEST,
        preferred_element_type=jnp.float32,
    )


def _lstm(x2d, wih_t, whh_t, b):
    G = 4 * EMBED
    return pl.pallas_call(
        _lstm_body,
        grid=(SEQ,),
        in_specs=[
            pl.BlockSpec((BATCH // 2, 128), lambda t: (t, 0)),
            pl.BlockSpec((EMBED, G), lambda t: (0, 0)),
            pl.BlockSpec((EMBED, G), lambda t: (0, 0)),
            pl.BlockSpec((1, G), lambda t: (0, 0)),
        ],
        out_specs=pl.BlockSpec((EMBED, BATCH), lambda t: (0, t)),
        out_shape=jax.ShapeDtypeStruct((EMBED, NTOK), jnp.float32),
        scratch_shapes=[
            pltpu.VMEM((BATCH, EMBED), jnp.float32),
            pltpu.VMEM((BATCH, EMBED), jnp.float32),
        ],
    )(x2d, wih_t, whh_t, b)


def kernel(samples, text, targets, in_embed, out_embed, W_ih, W_hh, b_ih, b_hh):
    E = in_embed.shape[1]
    sample_size = samples.shape[-1]

    txt_idx = text.reshape(-1).astype(jnp.int32)
    tgt_idx = targets.reshape(-1).astype(jnp.int32)
    samp_idx = jnp.transpose(samples, (2, 0, 1)).reshape(-1).astype(jnp.int32)

    txt_emb = _sc_gather_paired(in_embed, txt_idx, W=64, K=1)
    rnn_t = _lstm(
        txt_emb.reshape(NTOK // 2, 128),
        W_ih.T,
        W_hh.T,
        (b_ih + b_hh).reshape(1, -1),
    )
    tgt_emb = _sc_gather_paired(out_embed, tgt_idx, W=64, K=1)
    tgt_t = _tc_transpose(tgt_emb, 1, NTOK)
    samp_emb = _sc_gather_paired(out_embed, samp_idx, W=128, K=2)
    samp_t = _tc_transpose(samp_emb, sample_size, NTOK)

    return (
        jnp.transpose(samp_t, (2, 0, 1)),
        jnp.transpose(rnn_t, (1, 0))[:, :, None],
        jnp.transpose(tgt_t, (2, 0, 1)),
    )
